# baseline (speedup 1.0000x reference)
import functools

import jax
import jax.numpy as jnp
from jax import lax
from jax.experimental import pallas as pl
from jax.experimental.pallas import tpu as pltpu
from jax.experimental.pallas import tpu_sc as plsc

_B, _C, _H, _W = 4, 192, 384, 384
_J, _MH, _MW = 8, 96, 96
_FH, _FW = _H // _MH, _W // _MW
_CC = 32
_P = _MH * _MW
_NROW = _B * _C
_NTILE = 32
_RPT = _NROW // _NTILE
_Q = 4
_NCHUNK = _P // 16


def _pool_body(x_ref, xp_ref):
    pw = (jax.lax.broadcasted_iota(jnp.int32, (_W, _MW), 0) // _FW
          == jax.lax.broadcasted_iota(jnp.int32, (_W, _MW), 1)).astype(jnp.float32)
    y = jax.lax.dot_general(
        x_ref[0], pw, (((1,), (0,)), ((), ())), preferred_element_type=jnp.float32
    )
    sh = (jax.lax.broadcasted_iota(jnp.int32, (_MH, _H), 0)
          == jax.lax.broadcasted_iota(jnp.int32, (_MH, _H), 1) // _FH
          ).astype(jnp.float32)
    for c in range(_CC):
        yc = y[c * _H:(c + 1) * _H]
        xp_ref[0, c * _MH:(c + 1) * _MH, :] = jax.lax.dot_general(
            sh, yc, (((1,), (0,)), ((), ())), preferred_element_type=jnp.float32
        )


_sc_mesh = plsc.VectorSubcoreMesh(core_axis_name="c", subcore_axis_name="s")


@functools.partial(
    pl.kernel,
    mesh=_sc_mesh,
    out_type=jax.ShapeDtypeStruct((_NROW, _J * 16), jnp.float32),
    scratch_types=[
        pltpu.VMEM((_J, _P), jnp.float32),
        pltpu.VMEM((_Q, _P), jnp.float32),
        pltpu.VMEM((_RPT, _J * 16), jnp.float32),
    ],
)
def _sc_reduce(seg_hbm, xp_hbm, out_hbm, mask_v, rows_v, out_v):
    wid = lax.axis_index("s") * 2 + lax.axis_index("c")
    b = (wid * _RPT) // _C
    pltpu.sync_copy(seg_hbm.at[b], mask_v)
    for q in range(_RPT // _Q):
        pltpu.sync_copy(xp_hbm.at[pl.ds(wid * _RPT + q * _Q, _Q)], rows_v)

        def chunk_body(i, accs):
            base = i * 16
            mjs = [mask_v[j, pl.ds(base, 16)] for j in range(_J)]
            new = []
            for r in range(_Q):
                xv = rows_v[r, pl.ds(base, 16)]
                for j in range(_J):
                    new.append(accs[r * _J + j] + xv * mjs[j])
            return tuple(new)

        accs = lax.fori_loop(
            0, _NCHUNK, chunk_body,
            tuple(jnp.zeros((16,), jnp.float32) for _ in range(_Q * _J)),
        )
        for r in range(_Q):
            for j in range(_J):
                out_v[q * _Q + r, pl.ds(j * 16, 16)] = accs[r * _J + j]
    pltpu.sync_copy(out_v, out_hbm.at[pl.ds(wid * _RPT, _RPT)])


def _finish_body(sums_ref, seg_ref, wt_ref, b_ref, out_ref):
    seg = seg_ref[...]
    area = jnp.sum(jnp.where(seg != 0, 1.0, 0.0), axis=(2, 3)) * (_FH * _FW)
    for j in range(_J):
        s = jnp.concatenate(
            [jnp.sum(sums_ref[bi][:, j * 16:(j + 1) * 16], axis=1)[None, :]
             for bi in range(_B)], axis=0)
        a = area[:, j]
        feat = s / jnp.maximum(a, 1.0)[:, None]
        h = jnp.maximum(feat, 0.0)
        o = (
            jax.lax.dot_general(
                h, wt_ref[j], (((1,), (0,)), ((), ())),
                preferred_element_type=jnp.float32,
            )
            + b_ref[j][None, :]
        )
        o = jnp.where((a > 0)[:, None], o, 0.0)
        out_ref[:, j, :] = o


@jax.jit
def kernel(x, segmap_attentions, W, b):
    x3 = x.reshape(_B, _C * _H, _W)
    xp = pl.pallas_call(
        _pool_body,
        grid=(_B, _C // _CC),
        in_specs=[pl.BlockSpec((1, _CC * _H, _W), lambda b_, t: (b_, t, 0))],
        out_specs=pl.BlockSpec((1, _CC * _MH, _MW), lambda b_, t: (b_, t, 0)),
        out_shape=jax.ShapeDtypeStruct((_B, _C * _MH, _MW), jnp.float32),
        compiler_params=pltpu.CompilerParams(
            dimension_semantics=("parallel", "arbitrary"),
        ),
    )(x3)

    xp_rows = xp.reshape(_NROW, _P)
    seg_bin = (segmap_attentions != 0).astype(jnp.float32).reshape(_B, _J, _P)
    sums128 = _sc_reduce(seg_bin, xp_rows).reshape(_B, _C, _J * 16)

    wt = jnp.transpose(W, (0, 2, 1))
    out = pl.pallas_call(
        _finish_body,
        out_shape=jax.ShapeDtypeStruct((_B, _J, _C), jnp.float32),
    )(sums128, segmap_attentions, wt, b)
    return out
